# baseline (speedup 1.0000x reference)
import functools

import jax
import jax.numpy as jnp
from jax import lax
from jax.experimental import pallas as pl
from jax.experimental.pallas import tpu as pltpu
from jax.experimental.pallas import tpu_sc as plsc

VOCAB = 100000
EMB = 64
CTX = 20
BATCH = 4096

NUM_CORES = 2
NUM_SUBCORES = 16
NUM_WORKERS = NUM_CORES * NUM_SUBCORES
PB = BATCH // NUM_WORKERS

_sc_mesh = plsc.VectorSubcoreMesh(core_axis_name="c", subcore_axis_name="s")


@functools.partial(
    pl.kernel,
    mesh=_sc_mesh,
    out_type=jax.ShapeDtypeStruct((BATCH, EMB), jnp.float32),
    scratch_types=[
        pltpu.VMEM((PB,), jnp.int32),
        pltpu.VMEM((PB,), jnp.int32),
        pltpu.VMEM((PB, EMB), jnp.float32),
        pltpu.VMEM_SHARED((BATCH, EMB), jnp.float32),
        pltpu.SemaphoreType.DMA,
    ],
    compiler_params=pltpu.CompilerParams(use_tc_tiling_on_sc=False),
)
def _sc_pool(xt_hbm, ids_hbm, table_hbm, out_hbm, idx_v, dst_v, rows_v,
             acc_sh, sem):
    wid = lax.axis_index("s") * NUM_CORES + lax.axis_index("c")
    base = wid * PB
    pltpu.sync_copy(ids_hbm.at[pl.ds(base, PB)], dst_v)
    for j in range(CTX):
        pltpu.sync_copy(xt_hbm.at[pl.ds(j * BATCH + base, PB)], idx_v)
        pltpu.async_copy(table_hbm.at[idx_v], rows_v, sem).wait()
        if j == 0:
            pltpu.sync_copy(rows_v, acc_sh.at[dst_v])
        else:
            pltpu.sync_copy(rows_v, acc_sh.at[dst_v], add=True)
    pltpu.sync_copy(acc_sh.at[pl.ds(base, PB)], out_hbm.at[pl.ds(base, PB)])


BB = 1024
VB = 2048


def _mm_body(p_ref, w_ref, b_ref, o_ref):
    p = p_ref[...] * (1.0 / CTX)
    o_ref[...] = lax.dot_general(
        p, w_ref[...], (((1,), (1,)), ((), ())),
        preferred_element_type=jnp.float32) + b_ref[...]


def _tc_matmul(pooled, W, b2):
    return pl.pallas_call(
        _mm_body,
        grid=(BATCH // BB, pl.cdiv(VOCAB, VB)),
        in_specs=[
            pl.BlockSpec((BB, EMB), lambda i, j: (i, 0)),
            pl.BlockSpec((VB, EMB), lambda i, j: (j, 0)),
            pl.BlockSpec((1, VB), lambda i, j: (0, j)),
        ],
        out_specs=pl.BlockSpec((BB, VB), lambda i, j: (i, j)),
        out_shape=jax.ShapeDtypeStruct((BATCH, VOCAB), jnp.float32),
        compiler_params=pltpu.CompilerParams(
            dimension_semantics=("parallel", "parallel")),
    )(pooled, W, b2)


def kernel(x, emb_table, W, b):
    xt_flat = x.T.reshape(-1).astype(jnp.int32)
    ids = jnp.arange(BATCH, dtype=jnp.int32)
    pooled = _sc_pool(xt_flat, ids, emb_table)
    return _tc_matmul(pooled, W, b.reshape(1, VOCAB))
